# baseline (speedup 1.0000x reference)
import functools

import jax
import jax.numpy as jnp
from jax import lax
from jax.experimental import pallas as pl
from jax.experimental.pallas import tpu as pltpu
from jax.experimental.pallas import tpu_sc as plsc

B, P, D, L, C = 32, 784, 2048, 256, 100

_GDN = lax.GatherDimensionNumbers(
    offset_dims=(), collapsed_slice_dims=(0,), start_index_map=(0,)
)


def _vreg_take(vec, lanes):
    return lax.gather(
        vec,
        lanes[:, None],
        _GDN,
        slice_sizes=(1,),
        mode=lax.GatherScatterMode.PROMISE_IN_BOUNDS,
    )
NC, NS, LN = 2, 16, 16
NW = NC * NS
DW = D // NW
NJ = DW // LN
BBLK = 8


def _sc_body(x_hbm, pos_hbm, val_hbm, out_hbm, x_v, idx_v, pos_v, val_v, enc_v):
    c = lax.axis_index("c")
    s = lax.axis_index("s")
    wid = s * NC + c
    c0 = wid * DW

    pltpu.sync_copy(x_hbm, x_v)
    pltpu.sync_copy(pos_hbm.at[:, pl.ds(c0, DW)], pos_v)
    pltpu.sync_copy(val_hbm.at[:, pl.ds(c0, DW)], val_v)

    def qbody(i, _):
        v = x_v[pl.ds(i * LN, LN)] * jnp.float32(L - 1)
        t = v + jnp.float32(0.5)
        ii = t.astype(jnp.int32)
        tie = (ii.astype(jnp.float32) == t) & ((ii & 1) == 1)
        ii = jnp.where(tie, ii - 1, ii)
        ii = jnp.clip(ii, 0, L - 1)
        idx_v[pl.ds(i * LN, LN)] = ii
        return 0

    lax.fori_loop(0, B * P // LN, qbody, 0)

    iota = lax.iota(jnp.int32, LN)
    cols = [jnp.int32(j * LN) + iota for j in range(NJ)]

    zero = jnp.zeros((LN,), jnp.float32)
    for bb in range(B // BBLK):
        b_flat = (jnp.int32(bb * BBLK) + (iota & jnp.int32(BBLK - 1))) * jnp.int32(P)
        lane_sel = [jnp.full((LN,), si, jnp.int32) for si in range(BBLK)]

        def pbody(p, accs):
            idxrow = plsc.load_gather(idx_v, [b_flat + p])
            posr = [pos_v2[p, pl.ds(j * LN, LN)] for j in range(NJ)]
            out = []
            for si in range(BBLK):
                row = _vreg_take(idxrow, lane_sel[si])
                a = accs[si]
                a = tuple(
                    a[j] + plsc.load_gather(val_v2, [row, cols[j]]) * posr[j]
                    for j in range(NJ)
                )
                out.append(a)
            return tuple(out)

        pos_v2 = pos_v
        val_v2 = val_v
        accs = lax.fori_loop(
            0, P, pbody, tuple(tuple(zero for _ in range(NJ)) for _ in range(BBLK))
        )
        for si in range(BBLK):
            for j in range(NJ):
                e = jnp.where(accs[si][j] > 0, jnp.float32(1), jnp.float32(-1))
                enc_v[bb * BBLK + si, pl.ds(j * LN, LN)] = e

    pltpu.sync_copy(enc_v, out_hbm.at[:, pl.ds(c0, DW)])


@jax.jit
def _sc_encode(xf, position_weight, value_weight):
    mesh = plsc.VectorSubcoreMesh(core_axis_name="c", subcore_axis_name="s")
    f = functools.partial(
        pl.kernel,
        out_type=jax.ShapeDtypeStruct((B, D), jnp.float32),
        mesh=mesh,
        compiler_params=pltpu.CompilerParams(
            use_tc_tiling_on_sc=False, needs_layout_passes=False
        ),
        scratch_types=[
            pltpu.VMEM((B * P,), jnp.float32),
            pltpu.VMEM((B * P,), jnp.int32),
            pltpu.VMEM((P, DW), jnp.float32),
            pltpu.VMEM((L, DW), jnp.float32),
            pltpu.VMEM((B, DW), jnp.float32),
        ],
    )(_sc_body)
    return f(xf.reshape(B * P), position_weight, value_weight)


def _classify_body(enc_ref, w_ref, out_ref):
    out_ref[...] = lax.dot_general(
        enc_ref[...],
        w_ref[...],
        (((1,), (1,)), ((), ())),
        preferred_element_type=jnp.float32,
    )


@jax.jit
def _classify(enc, classify_weight):
    return pl.pallas_call(
        _classify_body,
        out_shape=jax.ShapeDtypeStruct((B, C), jnp.float32),
    )(enc, classify_weight)


def kernel(x, position_weight, value_weight, classify_weight):
    xf = x.reshape(B, P)
    enc = _sc_encode(xf, position_weight, value_weight)
    return _classify(enc, classify_weight)


# speedup vs baseline: 2.9946x; 2.9946x over previous
import functools

import jax
import jax.numpy as jnp
from jax import lax
from jax.experimental import pallas as pl
from jax.experimental.pallas import tpu as pltpu
from jax.experimental.pallas import tpu_sc as plsc

B, P, D, L, C = 32, 784, 2048, 256, 100
NC, NS, LN = 2, 16, 16
NW = NC * NS
DW = D // NW
WPT = DW // 4
BBLK = 8
SEG = 196
NSEG = P // SEG

_GDN = lax.GatherDimensionNumbers(
    offset_dims=(), collapsed_slice_dims=(0,), start_index_map=(0,)
)


def _vreg_take(vec, lanes):
    return lax.gather(
        vec,
        lanes[:, None],
        _GDN,
        slice_sizes=(1,),
        mode=lax.GatherScatterMode.PROMISE_IN_BOUNDS,
    )


def _pack_signs(w, rows):
    bits = (w < 0).astype(jnp.int8)
    bits = bits.reshape(rows, NW, 4, LN).transpose(0, 1, 3, 2)
    return lax.bitcast_convert_type(bits, jnp.int32).reshape(rows, D // 4)


def _sc_body(x_hbm, pos_hbm, val_hbm, out_hbm, x_v, idx_v, pos_v, val_v, enc_v, wacc_v):
    c = lax.axis_index("c")
    s = lax.axis_index("s")
    wid = s * NC + c
    c0 = wid * WPT

    pltpu.sync_copy(x_hbm, x_v)
    pltpu.sync_copy(pos_hbm.at[:, pl.ds(c0, WPT)], pos_v)
    pltpu.sync_copy(val_hbm.at[:, pl.ds(c0, WPT)], val_v)

    def qbody(i, _):
        v = x_v[pl.ds(i * LN, LN)] * jnp.float32(L - 1)
        t = v + jnp.float32(0.5)
        ii = t.astype(jnp.int32)
        tie = (ii.astype(jnp.float32) == t) & ((ii & 1) == 1)
        ii = jnp.where(tie, ii - 1, ii)
        ii = jnp.clip(ii, 0, L - 1)
        idx_v[pl.ds(i * LN, LN)] = ii
        return 0

    lax.fori_loop(0, B * P // LN, qbody, 0)

    iota = lax.iota(jnp.int32, LN)
    zero = jnp.zeros((LN,), jnp.int32)
    lane_sel = [jnp.full((LN,), si, jnp.int32) for si in range(BBLK)]
    byte_mask = jnp.full((LN,), 0xFF, jnp.int32)

    for bb in range(B // BBLK):
        b_flat = (jnp.int32(bb * BBLK) + (iota & jnp.int32(BBLK - 1))) * jnp.int32(P)
        for si in range(BBLK):
            for r in range(4):
                wacc_v[bb * BBLK + si, pl.ds(r * LN, LN)] = zero

        for seg in range(NSEG):

            def pbody(p, accs):
                idxrow = plsc.load_gather(idx_v, [b_flat + p])
                posw = pos_v[p, :]
                out = []
                for si in range(BBLK):
                    rbase = _vreg_take(idxrow, lane_sel[si])
                    valw = plsc.load_gather(val_v, [rbase, iota])
                    out.append(accs[si] + (valw ^ posw))
                return tuple(out)

            accs = lax.fori_loop(
                seg * SEG,
                (seg + 1) * SEG,
                pbody,
                tuple(zero for _ in range(BBLK)),
            )
            for si in range(BBLK):
                acc = accs[si]
                brow = bb * BBLK + si
                for r in range(4):
                    cnt = (lax.shift_right_logical(acc, jnp.int32(8 * r))
                           & byte_mask)
                    wacc_v[brow, pl.ds(r * LN, LN)] = (
                        wacc_v[brow, pl.ds(r * LN, LN)] + cnt
                    )

    half = jnp.int32(P // 2)
    for brow in range(B):
        for r in range(4):
            cnt = wacc_v[brow, pl.ds(r * LN, LN)]
            enc_v[brow, pl.ds(r * LN, LN)] = jnp.where(
                cnt < half, jnp.float32(1), jnp.float32(-1)
            )

    pltpu.sync_copy(enc_v, out_hbm.at[:, pl.ds(wid * DW, DW)])


@jax.jit
def _sc_encode(xf, pos_pk, val_pk):
    mesh = plsc.VectorSubcoreMesh(core_axis_name="c", subcore_axis_name="s")
    f = functools.partial(
        pl.kernel,
        out_type=jax.ShapeDtypeStruct((B, D), jnp.float32),
        mesh=mesh,
        compiler_params=pltpu.CompilerParams(
            use_tc_tiling_on_sc=False, needs_layout_passes=False
        ),
        scratch_types=[
            pltpu.VMEM((B * P,), jnp.float32),
            pltpu.VMEM((B * P,), jnp.int32),
            pltpu.VMEM((P, WPT), jnp.int32),
            pltpu.VMEM((L, WPT), jnp.int32),
            pltpu.VMEM((B, DW), jnp.float32),
            pltpu.VMEM((B, DW), jnp.int32),
        ],
    )(_sc_body)
    return f(xf.reshape(B * P), pos_pk, val_pk)


def _classify_body(enc_ref, w_ref, out_ref):
    out_ref[...] = lax.dot_general(
        enc_ref[...],
        w_ref[...],
        (((1,), (1,)), ((), ())),
        preferred_element_type=jnp.float32,
    )


@jax.jit
def _classify(enc, classify_weight):
    return pl.pallas_call(
        _classify_body,
        out_shape=jax.ShapeDtypeStruct((B, C), jnp.float32),
    )(enc, classify_weight)


def kernel(x, position_weight, value_weight, classify_weight):
    xf = x.reshape(B, P)
    pos_pk = _pack_signs(position_weight, P)
    val_pk = _pack_signs(value_weight, L)
    enc = _sc_encode(xf, pos_pk, val_pk)
    return _classify(enc, classify_weight)
